# speedup vs baseline: 5.7518x; 5.7518x over previous
import functools

import jax
import jax.numpy as jnp
from jax.experimental import pallas as pl

T = 1025
TA = 513
TB = 512
C = 96


def _tome_body(a_ref, b_ref, out_ref):
    A = a_ref[0]
    B = b_ref[0]
    na = A / jnp.sqrt(jnp.sum(A * A, axis=-1, keepdims=True))
    nb = B / jnp.sqrt(jnp.sum(B * B, axis=-1, keepdims=True))
    scores = jax.lax.dot_general(
        na, nb, (((1,), (1,)), ((), ())), preferred_element_type=jnp.float32
    )
    mx = jnp.max(scores, axis=-1, keepdims=True)
    lane = jax.lax.broadcasted_iota(jnp.int32, (TA, TB), 1)
    dst = jnp.min(jnp.where(scores == mx, lane, TB), axis=-1, keepdims=True)
    row = jax.lax.broadcasted_iota(jnp.int32, (TA, 1), 0)
    dst = jnp.where(row == 0, -1, dst)
    P = (lane == dst).astype(jnp.float32)
    A1 = jnp.concatenate([A, jnp.ones((TA, 1), jnp.float32)], axis=1)
    M = jax.lax.dot_general(
        P, A1, (((0,), (0,)), ((), ())), preferred_element_type=jnp.float32
    )
    merged = M[:, :C]
    counts = M[:, C : C + 1]
    dst_rows = (B + merged) / (1.0 + counts)
    out_ref[0] = jnp.concatenate([A[0:1, :], dst_rows], axis=0)


@functools.partial(jax.jit, static_argnames=("interpret",))
def _tome(a, b, interpret=False):
    nb_ = a.shape[0]
    return pl.pallas_call(
        _tome_body,
        grid=(nb_,),
        in_specs=[
            pl.BlockSpec((1, TA, C), lambda i: (i, 0, 0)),
            pl.BlockSpec((1, TB, C), lambda i: (i, 0, 0)),
        ],
        out_specs=pl.BlockSpec((1, TA, C), lambda i: (i, 0, 0)),
        out_shape=jax.ShapeDtypeStruct((nb_, TA, C), jnp.float32),
        interpret=interpret,
    )(a, b)


def kernel(hidden_states):
    a = hidden_states[:, ::2, :]
    b = hidden_states[:, 1::2, :]
    return _tome(a, b)


# speedup vs baseline: 7.5615x; 1.3146x over previous
import functools

import jax
import jax.numpy as jnp
from jax.experimental import pallas as pl

T = 1025
TA = 513
TB = 512
C = 96


BB = 4


def _tome_body(x_ref, out_ref):
    for bb in range(BB):
        _tome_one(x_ref, out_ref, bb)


def _tome_one(x_ref, out_ref, bb):
    A = x_ref[bb, pl.Slice(0, TA, 2), :]
    B = x_ref[bb, pl.Slice(1, TB, 2), :]
    na = A / jnp.sqrt(jnp.sum(A * A, axis=-1, keepdims=True))
    nb = B / jnp.sqrt(jnp.sum(B * B, axis=-1, keepdims=True))
    scores = jax.lax.dot_general(
        na, nb, (((1,), (1,)), ((), ())), preferred_element_type=jnp.float32
    )
    mx = jnp.max(scores, axis=-1, keepdims=True)
    lane = jax.lax.broadcasted_iota(jnp.int32, (TA, TB), 1)
    dst = jnp.min(jnp.where(scores == mx, lane, TB), axis=-1, keepdims=True)
    row = jax.lax.broadcasted_iota(jnp.int32, (TA, 1), 0)
    dst = jnp.where(row == 0, -1, dst)
    P = (lane == dst).astype(jnp.float32)
    A1 = jnp.concatenate([A, jnp.ones((TA, 1), jnp.float32)], axis=1)
    M = jax.lax.dot_general(
        P, A1, (((0,), (0,)), ((), ())), preferred_element_type=jnp.float32
    )
    merged = M[:, :C]
    counts = M[:, C : C + 1]
    dst_rows = (B + merged) / (1.0 + counts)
    out_ref[0] = jnp.concatenate([A[0:1, :], dst_rows], axis=0)


@functools.partial(jax.jit, static_argnames=("interpret",))
def _tome(x, interpret=False):
    nb_ = x.shape[0]
    return pl.pallas_call(
        _tome_body,
        grid=(nb_ // BB,),
        in_specs=[
            pl.BlockSpec((BB, T, C), lambda i: (i, 0, 0)),
        ],
        out_specs=pl.BlockSpec((BB, TA, C), lambda i: (i, 0, 0)),
        out_shape=jax.ShapeDtypeStruct((nb_, TA, C), jnp.float32),
        interpret=interpret,
    )(x)


def kernel(hidden_states):
    return _tome(hidden_states)
